# baseline (speedup 1.0000x reference)
import math

import numpy as np
import jax
import jax.numpy as jnp
from jax import lax
from jax.experimental import pallas as pl
from jax.experimental.pallas import tpu as pltpu
from jax.experimental.pallas import tpu_sc as plsc

VOCAB = 100000
D_MODEL = 768
SEQ = 4096
BATCH = 4
SCALE = math.sqrt(D_MODEL)

NC, NS = 2, 16
NW = NC * NS
P_PER_W = SEQ // NW
KP = 32
NPC = P_PER_W // KP
NSTEP = NPC * BATCH
LANES = 16
VECS = D_MODEL // LANES
NOB = 3


def _make_pe(seq_len: int, d_model: int) -> np.ndarray:
    position = np.arange(0, seq_len, dtype=np.float32)[:, None]
    div_term = np.exp(
        np.arange(0, d_model, 2).astype(np.float32) * (-math.log(10000.0) / d_model)
    )
    pe = np.zeros((seq_len, d_model), dtype=np.float32)
    pe[:, 0::2] = np.sin(position * div_term)
    pe[:, 1::2] = np.cos(position * div_term)
    return pe


_PE = _make_pe(SEQ, D_MODEL)


def _body(tok_hbm, table_hbm, pe_hbm, out_hbm,
          idx_v, r0, r1, o0, o1, o2,
          gs0, gs1, ps0, ps1, ps2, os0, os1, os2):
    rbuf = (r0, r1)
    obuf = (o0, o1, o2)
    gsem = (gs0, gs1)
    psem = (ps0, ps1, ps2)
    osem = (os0, os1, os2)

    c = lax.axis_index("c")
    s = lax.axis_index("s")
    wid = s * NC + c
    posbase = wid * P_PER_W

    pltpu.sync_copy(tok_hbm.at[:, pl.ds(posbase, P_PER_W)], idx_v)

    def step_pcb(k):
        return divmod(k, BATCH)

    def start_pe(k):
        pc, _ = step_pcb(k)
        return pltpu.async_copy(
            pe_hbm.at[pl.ds(posbase + pc * KP, KP)], obuf[k % NOB], psem[k % NOB])

    def start_gather(k):
        pc, b = step_pcb(k)
        return pltpu.async_copy(
            table_hbm.at[idx_v.at[b, pl.ds(pc * KP, KP)]], rbuf[k % 2],
            gsem[k % 2])

    def start_out(k):
        pc, b = step_pcb(k)
        dst = out_hbm.at[pl.ds(b * SEQ + posbase + pc * KP, KP)]
        return pltpu.async_copy(obuf[k % NOB], dst, osem[k % NOB])

    pend_pe = [start_pe(0), None, None]
    pend_g = [start_gather(0), start_gather(1)]
    pend_out = [None, None, None]

    for k in range(NSTEP):
        gb = k % 2
        ob = k % NOB

        kn = k + 1
        if kn < NSTEP:
            nob = kn % NOB
            if pend_out[nob] is not None:
                pend_out[nob].wait()
                pend_out[nob] = None
            pend_pe[nob] = start_pe(kn)

        pend_g[gb].wait()
        pend_pe[ob].wait()
        pend_pe[ob] = None

        rv = rbuf[gb]
        ov = obuf[ob]

        @plsc.parallel_loop(0, KP, 1, unroll=2)
        def _row_acc(r):
            for j in range(VECS):
                sl = pl.ds(j * LANES, LANES)
                plsc.addupdate(ov.at[r, sl], rv[r, sl] * SCALE)

        pend_out[ob] = start_out(k)

        if k + 2 < NSTEP:
            pend_g[gb] = start_gather(k + 2)

    for d in pend_out:
        if d is not None:
            d.wait()


def kernel(tokens, table):
    pe = jnp.asarray(_PE)
    mesh = plsc.VectorSubcoreMesh(core_axis_name="c", subcore_axis_name="s")
    out = pl.kernel(
        _body,
        out_type=jax.ShapeDtypeStruct((BATCH * SEQ, D_MODEL), jnp.float32),
        mesh=mesh,
        scratch_types=[
            pltpu.VMEM((BATCH, P_PER_W), jnp.int32),
            pltpu.VMEM((KP, D_MODEL), jnp.float32),
            pltpu.VMEM((KP, D_MODEL), jnp.float32),
            pltpu.VMEM((KP, D_MODEL), jnp.float32),
            pltpu.VMEM((KP, D_MODEL), jnp.float32),
            pltpu.VMEM((KP, D_MODEL), jnp.float32),
            pltpu.SemaphoreType.DMA,
            pltpu.SemaphoreType.DMA,
            pltpu.SemaphoreType.DMA,
            pltpu.SemaphoreType.DMA,
            pltpu.SemaphoreType.DMA,
            pltpu.SemaphoreType.DMA,
            pltpu.SemaphoreType.DMA,
            pltpu.SemaphoreType.DMA,
        ],
    )(tokens.astype(jnp.int32), table, pe)
    return out.reshape(BATCH, SEQ, D_MODEL)


# speedup vs baseline: 1.2323x; 1.2323x over previous
import math

import numpy as np
import jax
import jax.numpy as jnp
from jax import lax
from jax.experimental import pallas as pl
from jax.experimental.pallas import tpu as pltpu
from jax.experimental.pallas import tpu_sc as plsc

VOCAB = 100000
D_MODEL = 768
SEQ = 4096
BATCH = 4
SCALE = math.sqrt(D_MODEL)

NC, NS = 2, 16
NW = NC * NS
P_PER_W = SEQ // NW
KP = 16
HALF = KP // 2
NPC = P_PER_W // KP
LANES = 16
VECS = D_MODEL // LANES


def _make_pe(seq_len: int, d_model: int) -> np.ndarray:
    position = np.arange(0, seq_len, dtype=np.float32)[:, None]
    div_term = np.exp(
        np.arange(0, d_model, 2).astype(np.float32) * (-math.log(10000.0) / d_model)
    )
    pe = np.zeros((seq_len, d_model), dtype=np.float32)
    pe[:, 0::2] = np.sin(position * div_term)
    pe[:, 1::2] = np.cos(position * div_term)
    return pe


_PE = _make_pe(SEQ, D_MODEL)


def _body(tok_hbm, table_hbm, pe_hbm, out_hbm,
          idx_v,
          rb00, rb01, rb02, rb03, rb10, rb11, rb12, rb13,
          pv0, pv1,
          g00, g01, g02, g03, g10, g11, g12, g13,
          o00, o01, o02, o03, o10, o11, o12, o13,
          p0, p1):
    rbuf = ((rb00, rb01, rb02, rb03), (rb10, rb11, rb12, rb13))
    pv = (pv0, pv1)
    gsem = ((g00, g01, g02, g03), (g10, g11, g12, g13))
    osem = ((o00, o01, o02, o03), (o10, o11, o12, o13))
    psem = (p0, p1)

    c = lax.axis_index("c")
    s = lax.axis_index("s")
    wid = s * NC + c
    posbase = wid * P_PER_W

    pltpu.sync_copy(tok_hbm.at[:, pl.ds(posbase, P_PER_W)], idx_v)

    def start_pe(pc):
        return pltpu.async_copy(
            pe_hbm.at[pl.ds(posbase + pc * KP, KP)], pv[pc % 2], psem[pc % 2])

    def start_gather(pc, b):
        return pltpu.async_copy(
            table_hbm.at[idx_v.at[b, pl.ds(pc * KP, KP)]], rbuf[pc % 2][b],
            gsem[pc % 2][b])

    def start_out(pc, b):
        dst = out_hbm.at[pl.ds(b * SEQ + posbase + pc * KP, KP)]
        return pltpu.async_copy(rbuf[pc % 2][b], dst, osem[pc % 2][b])

    def compute_rows(rvs, pvv, r_lo, r_hi):
        def row_body(r, carry):
            @plsc.parallel_loop(0, VECS, 1, unroll=8)
            def _col(j):
                sl = pl.ds(j * LANES, LANES)
                p = pvv[r, sl]
                for b in range(BATCH):
                    rvs[b][r, sl] = rvs[b][r, sl] * SCALE + p
            return carry
        lax.fori_loop(r_lo, r_hi, row_body, 0)

    pend_pe = [start_pe(0), start_pe(1)]
    pend_g = [[start_gather(0, b) for b in range(BATCH)], [None] * BATCH]
    pend_out = [[None] * BATCH, [None] * BATCH]

    for pc in range(NPC):
        cs = pc % 2
        ns = 1 - cs
        for b in range(BATCH):
            pend_g[cs][b].wait()
        pend_pe[cs].wait()

        rvs = rbuf[cs]
        pvv = pv[cs]

        compute_rows(rvs, pvv, 0, HALF)

        if pc + 1 < NPC:
            for b in range(BATCH):
                if pend_out[ns][b] is not None:
                    pend_out[ns][b].wait()
                    pend_out[ns][b] = None
                pend_g[ns][b] = start_gather(pc + 1, b)

        compute_rows(rvs, pvv, HALF, KP)

        for b in range(BATCH):
            pend_out[cs][b] = start_out(pc, b)
        if pc + 2 < NPC:
            pend_pe[cs] = start_pe(pc + 2)

    for side in pend_out:
        for d in side:
            if d is not None:
                d.wait()


def kernel(tokens, table):
    pe = jnp.asarray(_PE)
    mesh = plsc.VectorSubcoreMesh(core_axis_name="c", subcore_axis_name="s")
    buf = pltpu.VMEM((KP, D_MODEL), jnp.float32)
    out = pl.kernel(
        _body,
        out_type=jax.ShapeDtypeStruct((BATCH * SEQ, D_MODEL), jnp.float32),
        mesh=mesh,
        scratch_types=(
            [pltpu.VMEM((BATCH, P_PER_W), jnp.int32)]
            + [buf] * 10
            + [pltpu.SemaphoreType.DMA] * 18
        ),
    )(tokens.astype(jnp.int32), table, pe)
    return out.reshape(BATCH, SEQ, D_MODEL)
